# baseline (speedup 1.0000x reference)
import functools

import jax
import jax.numpy as jnp
from jax import lax
from jax.experimental import pallas as pl
from jax.experimental.pallas import tpu as pltpu
from jax.experimental.pallas import tpu_sc as plsc

R = 32
UNROLL = 8


def kernel(x, pos_emb):
    B, T, C = x.shape
    info = plsc.get_sparse_core_info()
    NW = info.num_cores * info.num_subcores
    tw = T // NW
    nchunks = tw // R
    nitems = nchunks * B
    CH = R * C

    xf = x.reshape(B * T * C)
    pef = pos_emb[:T].reshape(T * C)
    mesh = plsc.VectorSubcoreMesh(core_axis_name="c", subcore_axis_name="s")

    @functools.partial(
        pl.kernel,
        mesh=mesh,
        out_type=jax.ShapeDtypeStruct((B * T * C,), jnp.float32),
        scratch_types=[
            pltpu.VMEM((2, CH), jnp.float32),
            pltpu.VMEM((CH,), jnp.float32),
            pltpu.SemaphoreType.DMA,
            pltpu.SemaphoreType.DMA,
            pltpu.SemaphoreType.DMA,
        ],
    )
    def k(x_hbm, pe_hbm, out_hbm, xb, peb, xsem, pesem, osem):
        wid = lax.axis_index("s") * info.num_cores + lax.axis_index("c")
        t0 = wid * tw

        def xoff(item):
            return ((item % B) * T + t0 + (item // B) * R) * C

        def peoff(i):
            return (t0 + i * R) * C

        pltpu.async_copy(pe_hbm.at[pl.ds(peoff(0), CH)], peb, pesem)
        pltpu.async_copy(x_hbm.at[pl.ds(xoff(0), CH)], xb.at[0], xsem)

        def body(kk, _):
            i = kk // B
            b = kk % B
            slot = kk % 2
            nslot = (kk + 1) % 2

            @pl.when(kk >= 1)
            def _():
                pltpu.make_async_copy(
                    xb.at[nslot], out_hbm.at[pl.ds(xoff(kk - 1), CH)], osem
                ).wait()

            @pl.when(kk + 1 < nitems)
            def _():
                pltpu.async_copy(
                    x_hbm.at[pl.ds(xoff(kk + 1), CH)], xb.at[nslot], xsem
                )

            pltpu.make_async_copy(
                x_hbm.at[pl.ds(xoff(kk), CH)], xb.at[slot], xsem
            ).wait()

            @pl.when(b == 0)
            def _():
                pltpu.make_async_copy(
                    pe_hbm.at[pl.ds(peoff(i), CH)], peb, pesem
                ).wait()

            def add_body(j, _):
                base = j * (16 * UNROLL)
                for u in range(UNROLL):
                    s = pl.ds(base + u * 16, 16)
                    plsc.addupdate(xb.at[slot, s], peb[s])
                return 0

            lax.fori_loop(0, CH // (16 * UNROLL), add_body, 0)

            @pl.when((b == B - 1) & (i + 1 < nchunks))
            def _():
                pltpu.async_copy(
                    pe_hbm.at[pl.ds(peoff(i + 1), CH)], peb, pesem
                )

            pltpu.async_copy(
                xb.at[slot], out_hbm.at[pl.ds(xoff(kk), CH)], osem
            )
            return 0

        lax.fori_loop(0, nitems, body, 0)
        pltpu.make_async_copy(
            xb.at[(nitems - 1) % 2],
            out_hbm.at[pl.ds(xoff(nitems - 1), CH)],
            osem,
        ).wait()

    return k(xf, pef).reshape(B, T, C)


# speedup vs baseline: 3.9575x; 3.9575x over previous
import functools

import jax
import jax.numpy as jnp
from jax import lax
from jax.experimental import pallas as pl
from jax.experimental.pallas import tpu as pltpu
from jax.experimental.pallas import tpu_sc as plsc

R = 32
UNROLL = 8


def kernel(x, pos_emb):
    B, T, C = x.shape
    info = plsc.get_sparse_core_info()
    NW = info.num_cores * info.num_subcores
    tw = T // NW
    nchunks = tw // R
    nitems = nchunks * B
    CPG = C // 16

    mesh = plsc.VectorSubcoreMesh(core_axis_name="c", subcore_axis_name="s")

    @functools.partial(
        pl.kernel,
        mesh=mesh,
        out_type=jax.ShapeDtypeStruct((B, T, C), jnp.float32),
        scratch_types=[
            pltpu.VMEM((2, R, C), jnp.float32),
            pltpu.VMEM((R, C), jnp.float32),
            pltpu.SemaphoreType.DMA,
            pltpu.SemaphoreType.DMA,
            pltpu.SemaphoreType.DMA,
        ],
    )
    def k(x_hbm, pe_hbm, out_hbm, xb, peb, xsem, pesem, osem):
        wid = lax.axis_index("s") * info.num_cores + lax.axis_index("c")
        t0 = wid * tw

        def trow(item):
            return t0 + (item // B) * R

        pltpu.async_copy(pe_hbm.at[pl.ds(t0, R)], peb, pesem)
        pltpu.async_copy(x_hbm.at[0, pl.ds(t0, R)], xb.at[0], xsem)

        def body(kk, _):
            b = kk % B
            slot = kk % 2
            nslot = (kk + 1) % 2

            @pl.when(kk >= 1)
            def _():
                pltpu.make_async_copy(
                    xb.at[nslot],
                    out_hbm.at[(kk - 1) % B, pl.ds(trow(kk - 1), R)],
                    osem,
                ).wait()

            @pl.when(kk + 1 < nitems)
            def _():
                pltpu.async_copy(
                    x_hbm.at[(kk + 1) % B, pl.ds(trow(kk + 1), R)],
                    xb.at[nslot],
                    xsem,
                )

            pltpu.make_async_copy(
                x_hbm.at[b, pl.ds(trow(kk), R)], xb.at[slot], xsem
            ).wait()

            @pl.when(b == 0)
            def _():
                pltpu.make_async_copy(
                    pe_hbm.at[pl.ds(trow(kk), R)], peb, pesem
                ).wait()

            @plsc.parallel_loop(0, R * CPG, unroll=UNROLL)
            def _(g):
                row = g // CPG
                col = (g % CPG) * 16
                plsc.addupdate(
                    xb.at[slot, row, pl.ds(col, 16)],
                    peb[row, pl.ds(col, 16)],
                )

            @pl.when((b == B - 1) & (kk + 1 < nitems))
            def _():
                pltpu.async_copy(
                    pe_hbm.at[pl.ds(trow(kk + 1), R)], peb, pesem
                )

            pltpu.async_copy(
                xb.at[slot], out_hbm.at[b, pl.ds(trow(kk), R)], osem
            )
            return 0

        lax.fori_loop(0, nitems, body, 0)
        pltpu.make_async_copy(
            xb.at[(nitems - 1) % 2],
            out_hbm.at[(nitems - 1) % B, pl.ds(trow(nitems - 1), R)],
            osem,
        ).wait()

    return k(x, pos_emb[:T])


# speedup vs baseline: 4.8395x; 1.2229x over previous
import functools

import jax
import jax.numpy as jnp
from jax import lax
from jax.experimental import pallas as pl
from jax.experimental.pallas import tpu as pltpu
from jax.experimental.pallas import tpu_sc as plsc

R = 16
NBUF = 4
UNROLL = 8


def kernel(x, pos_emb):
    B, T, C = x.shape
    info = plsc.get_sparse_core_info()
    NW = info.num_cores * info.num_subcores
    tw = T // NW
    nchunks = tw // R
    nitems = nchunks * B
    CPG = C // 16

    mesh = plsc.VectorSubcoreMesh(core_axis_name="c", subcore_axis_name="s")

    @functools.partial(
        pl.kernel,
        mesh=mesh,
        out_type=jax.ShapeDtypeStruct((B, T, C), jnp.float32),
        scratch_types=[
            pltpu.VMEM((NBUF, R, C), jnp.float32),
            pltpu.VMEM((2, R, C), jnp.float32),
            pltpu.SemaphoreType.DMA,
            pltpu.SemaphoreType.DMA,
            pltpu.SemaphoreType.DMA,
        ],
    )
    def k(x_hbm, pe_hbm, out_hbm, xb, peb, xsem, pesem, osem):
        wid = lax.axis_index("s") * info.num_cores + lax.axis_index("c")
        t0 = wid * tw

        def trow(item):
            return t0 + (item // B) * R

        pltpu.async_copy(pe_hbm.at[pl.ds(t0, R)], peb.at[0], pesem)
        pltpu.async_copy(pe_hbm.at[pl.ds(t0 + R, R)], peb.at[1], pesem)
        pltpu.async_copy(x_hbm.at[0, pl.ds(t0, R)], xb.at[0], xsem)
        pltpu.async_copy(x_hbm.at[1, pl.ds(t0, R)], xb.at[1], xsem)

        def body(kk, _):
            i = kk // B
            b = kk % B
            slot = kk % NBUF
            pslot = i % 2

            @pl.when(kk >= 2)
            def _():
                pltpu.make_async_copy(
                    xb.at[(kk - 2) % NBUF],
                    out_hbm.at[(kk - 2) % B, pl.ds(trow(kk - 2), R)],
                    osem,
                ).wait()

            @pl.when(kk + 2 < nitems)
            def _():
                pltpu.async_copy(
                    x_hbm.at[(kk + 2) % B, pl.ds(trow(kk + 2), R)],
                    xb.at[(kk + 2) % NBUF],
                    xsem,
                )

            pltpu.make_async_copy(
                x_hbm.at[b, pl.ds(trow(kk), R)], xb.at[slot], xsem
            ).wait()

            @pl.when(b == 0)
            def _():
                pltpu.make_async_copy(
                    pe_hbm.at[pl.ds(trow(kk), R)], peb.at[pslot], pesem
                ).wait()

            @plsc.parallel_loop(0, R * CPG, unroll=UNROLL)
            def _(g):
                row = g // CPG
                col = (g % CPG) * 16
                plsc.addupdate(
                    xb.at[slot, row, pl.ds(col, 16)],
                    peb[pslot, row, pl.ds(col, 16)],
                )

            @pl.when((b == B - 1) & (i + 2 < nchunks))
            def _():
                pltpu.async_copy(
                    pe_hbm.at[pl.ds(t0 + (i + 2) * R, R)], peb.at[pslot], pesem
                )

            pltpu.async_copy(
                xb.at[slot], out_hbm.at[b, pl.ds(trow(kk), R)], osem
            )
            return 0

        lax.fori_loop(0, nitems, body, 0)
        pltpu.make_async_copy(
            xb.at[(nitems - 2) % NBUF],
            out_hbm.at[(nitems - 2) % B, pl.ds(trow(nitems - 2), R)],
            osem,
        ).wait()
        pltpu.make_async_copy(
            xb.at[(nitems - 1) % NBUF],
            out_hbm.at[(nitems - 1) % B, pl.ds(trow(nitems - 1), R)],
            osem,
        ).wait()

    return k(x, pos_emb[:T])
